# Initial kernel scaffold; baseline (speedup 1.0000x reference)
import jax
import jax.numpy as jnp
from jax.experimental import pallas as pl


def kernel(node_logits, edge_logits, knode, kedge):
    raise NotImplementedError("write your pallas kernel here")


# speedup vs baseline: 4.3001x; 4.3001x over previous
import functools

import jax
import jax.numpy as jnp
from jax import lax
from jax.experimental import pallas as pl
from jax.experimental.pallas import tpu as pltpu
from jax.experimental.pallas import tpu_sc as plsc

K_SEL = 5
NC = 2
NS = 16
NW = NC * NS
LANES = 16
IDX_W = 8


def _make_sc_topk(npad, c_dim, r_sub):
    groups = r_sub // LANES
    mesh = plsc.VectorSubcoreMesh(core_axis_name="c", subcore_axis_name="s")

    @functools.partial(
        pl.kernel,
        out_type=jax.ShapeDtypeStruct((npad * IDX_W,), jnp.int32),
        mesh=mesh,
        scratch_types=[
            pltpu.VMEM((r_sub * c_dim,), jnp.float32),
            pltpu.VMEM((r_sub * IDX_W,), jnp.int32),
        ],
        compiler_params=pltpu.CompilerParams(
            needs_layout_passes=False, use_tc_tiling_on_sc=False),
    )
    def sc_topk(logits_hbm, idx_hbm, buf_v, idx_v):
        wid = lax.axis_index("s") * NC + lax.axis_index("c")
        base = wid * r_sub
        pltpu.sync_copy(logits_hbm.at[pl.ds(base * c_dim, r_sub * c_dim)], buf_v)
        iota = lax.iota(jnp.int32, LANES)
        neg_inf = jnp.full((LANES,), -jnp.inf, dtype=jnp.float32)
        zeros_i = jnp.zeros((LANES,), jnp.int32)

        def group_body(g, carry):
            fb = g * (LANES * c_dim)
            rowcol = (g * LANES + iota) * IDX_W

            for j in range(K_SEL):
                def c_body(ci, mi):
                    m, idx = mi
                    v = plsc.load_gather(buf_v, [fb + iota * c_dim + ci])
                    take = v > m
                    return (jnp.where(take, v, m),
                            jnp.where(take, jnp.broadcast_to(ci, (LANES,)), idx))

                m, idx = lax.fori_loop(0, c_dim, c_body, (neg_inf, zeros_i))
                plsc.store_scatter(buf_v, [fb + iota * c_dim + idx], neg_inf)
                plsc.store_scatter(idx_v, [rowcol + j], idx)
            return carry

        lax.fori_loop(0, groups, group_body, 0)
        pltpu.sync_copy(idx_v, idx_hbm.at[pl.ds(base * IDX_W, r_sub * IDX_W)])

    return sc_topk


def _make_tc_onehot_mm(npad, c_dim, rb):
    nb = npad // rb

    def body(idx_ref, tab_ref, out_ref):
        iota_c = lax.broadcasted_iota(jnp.int32, (rb, c_dim), 1)
        mask = jnp.zeros((rb, c_dim), jnp.float32)
        for j in range(K_SEL):
            mask += (iota_c == idx_ref[:, j:j + 1]).astype(jnp.float32)
        out_ref[...] = jnp.dot(mask, tab_ref[...],
                               preferred_element_type=jnp.float32)

    return pl.pallas_call(
        body,
        grid=(nb,),
        in_specs=[
            pl.BlockSpec((rb, IDX_W), lambda i: (i, 0)),
            pl.BlockSpec((c_dim, 512), lambda i: (0, 0)),
        ],
        out_specs=pl.BlockSpec((rb, 512), lambda i: (i, 0)),
        out_shape=jax.ShapeDtypeStruct((npad, 512), jnp.float32),
    )


def _select(logits, table, npad, r_sub, rb):
    n, c_dim = logits.shape
    flat = jnp.pad(logits, ((0, npad - n), (0, 0))).reshape(-1)
    idx = _make_sc_topk(npad, c_dim, r_sub)(flat).reshape(npad, IDX_W)
    out = _make_tc_onehot_mm(npad, c_dim, rb)(idx, table)
    return out[:n]


def kernel(node_logits, edge_logits, knode, kedge):
    knode_out = _select(node_logits, knode, npad=512, r_sub=16, rb=512)
    kedge_out = _select(edge_logits, kedge, npad=90112, r_sub=2816, rb=512)
    return (knode_out, kedge_out)


# speedup vs baseline: 5.4355x; 1.2641x over previous
import functools

import jax
import jax.numpy as jnp
from jax import lax
from jax.experimental import pallas as pl
from jax.experimental.pallas import tpu as pltpu
from jax.experimental.pallas import tpu_sc as plsc

K_SEL = 5
NC = 2
NS = 16
NW = NC * NS
LANES = 16
IDX_W = 8


def _make_sc_topk(npad, c_dim, r_sub):
    groups = r_sub // LANES
    mesh = plsc.VectorSubcoreMesh(core_axis_name="c", subcore_axis_name="s")

    @functools.partial(
        pl.kernel,
        out_type=jax.ShapeDtypeStruct((npad * IDX_W,), jnp.int32),
        mesh=mesh,
        scratch_types=[
            pltpu.VMEM((r_sub * c_dim,), jnp.float32),
            pltpu.VMEM((r_sub * IDX_W,), jnp.int32),
        ],
        compiler_params=pltpu.CompilerParams(
            needs_layout_passes=False, use_tc_tiling_on_sc=False),
    )
    def sc_topk(logits_hbm, idx_hbm, buf_v, idx_v):
        wid = lax.axis_index("s") * NC + lax.axis_index("c")
        base = wid * r_sub
        pltpu.sync_copy(logits_hbm.at[pl.ds(base * c_dim, r_sub * c_dim)], buf_v)
        iota = lax.iota(jnp.int32, LANES)
        neg_inf = jnp.full((LANES,), -jnp.inf, dtype=jnp.float32)
        zeros_i = jnp.zeros((LANES,), jnp.int32)

        def group_body(g, carry):
            fb = g * (LANES * c_dim)
            rowcol = (g * LANES + iota) * IDX_W

            for j in range(K_SEL):
                def c_body(ci, mi):
                    m, idx = mi
                    v = plsc.load_gather(buf_v, [fb + iota * c_dim + ci])
                    take = v > m
                    return (jnp.where(take, v, m),
                            jnp.where(take, jnp.broadcast_to(ci, (LANES,)), idx))

                m, idx = lax.fori_loop(0, c_dim, c_body, (neg_inf, zeros_i))
                plsc.store_scatter(buf_v, [fb + iota * c_dim + idx], neg_inf)
                plsc.store_scatter(idx_v, [rowcol + j], idx)
            return carry

        lax.fori_loop(0, groups, group_body, 0)
        pltpu.sync_copy(idx_v, idx_hbm.at[pl.ds(base * IDX_W, r_sub * IDX_W)])

    return sc_topk


def _make_tc_onehot_mm(n_out, npad, c_dim, rb):
    nb = npad // rb

    def body(idx_ref, tab_ref, out_ref):
        iota_c = lax.broadcasted_iota(jnp.int32, (rb, c_dim), 1)
        mask = jnp.zeros((rb, c_dim), jnp.float32)
        for j in range(K_SEL):
            mask += (iota_c == idx_ref[:, j:j + 1]).astype(jnp.float32)
        out_ref[...] = jnp.dot(mask, tab_ref[...],
                               preferred_element_type=jnp.float32)

    return pl.pallas_call(
        body,
        grid=(nb,),
        in_specs=[
            pl.BlockSpec((rb, IDX_W), lambda i: (i, 0)),
            pl.BlockSpec((c_dim, 512), lambda i: (0, 0)),
        ],
        out_specs=pl.BlockSpec((rb, 512), lambda i: (i, 0)),
        out_shape=jax.ShapeDtypeStruct((n_out, 512), jnp.float32),
    )


def _select(logits, table, npad, r_sub, rb):
    n, c_dim = logits.shape
    flat = jnp.pad(logits, ((0, npad - n), (0, 0))).reshape(-1)
    idx = _make_sc_topk(npad, c_dim, r_sub)(flat).reshape(npad, IDX_W)
    return _make_tc_onehot_mm(n, npad, c_dim, rb)(idx, table)


def kernel(node_logits, edge_logits, knode, kedge):
    knode_out = _select(node_logits, knode, npad=512, r_sub=16, rb=512)
    kedge_out = _select(edge_logits, kedge, npad=90112, r_sub=2816, rb=512)
    return (knode_out, kedge_out)


# speedup vs baseline: 6.9847x; 1.2850x over previous
import functools

import jax
import jax.numpy as jnp
from jax import lax
from jax.experimental import pallas as pl
from jax.experimental.pallas import tpu as pltpu
from jax.experimental.pallas import tpu_sc as plsc

K_SEL = 5
NC = 2
NS = 16
NW = NC * NS
LANES = 16
IDX_W = 8


def _make_sc_topk(npad, c_dim, r_sub):
    groups = r_sub // LANES
    mesh = plsc.VectorSubcoreMesh(core_axis_name="c", subcore_axis_name="s")

    @functools.partial(
        pl.kernel,
        out_type=jax.ShapeDtypeStruct((npad * IDX_W,), jnp.int32),
        mesh=mesh,
        scratch_types=[
            pltpu.VMEM((r_sub * c_dim,), jnp.float32),
            pltpu.VMEM((r_sub * IDX_W,), jnp.int32),
        ],
        compiler_params=pltpu.CompilerParams(
            needs_layout_passes=False, use_tc_tiling_on_sc=False),
    )
    def sc_topk(logits_hbm, idx_hbm, buf_v, idx_v):
        wid = lax.axis_index("s") * NC + lax.axis_index("c")
        base = wid * r_sub
        pltpu.sync_copy(logits_hbm.at[pl.ds(base * c_dim, r_sub * c_dim)], buf_v)
        iota = lax.iota(jnp.int32, LANES)
        neg_inf = jnp.full((LANES,), -jnp.inf, dtype=jnp.float32)
        zeros_i = jnp.zeros((LANES,), jnp.int32)

        if c_dim <= 32:
            def group_body(g, carry):
                fb = g * (LANES * c_dim)
                rowcol = (g * LANES + iota) * IDX_W
                vals = [plsc.load_gather(buf_v, [fb + iota * c_dim + c])
                        for c in range(c_dim)]

                for j in range(K_SEL):
                    nodes = []
                    for c in range(0, c_dim - 1, 2):
                        take = vals[c + 1] > vals[c]
                        nodes.append((jnp.where(take, vals[c + 1], vals[c]),
                                      jnp.where(take, jnp.int32(c + 1),
                                                jnp.int32(c))))
                    if c_dim % 2:
                        nodes.append((vals[c_dim - 1],
                                      jnp.broadcast_to(jnp.int32(c_dim - 1),
                                                       (LANES,))))
                    while len(nodes) > 1:
                        nxt = []
                        for i in range(0, len(nodes) - 1, 2):
                            (va, ia), (vb, ib) = nodes[i], nodes[i + 1]
                            take = vb > va
                            nxt.append((jnp.where(take, vb, va),
                                        jnp.where(take, ib, ia)))
                        if len(nodes) % 2:
                            nxt.append(nodes[-1])
                        nodes = nxt
                    _, idx = nodes[0]
                    plsc.store_scatter(idx_v, [rowcol + j], idx)
                    if j < K_SEL - 1:
                        vals = [jnp.where(idx == c, neg_inf, vals[c])
                                for c in range(c_dim)]
                return carry
        else:
            def group_body(g, carry):
                fb = g * (LANES * c_dim)
                rowcol = (g * LANES + iota) * IDX_W

                for j in range(K_SEL):
                    def c_body(ci, mi):
                        m, idx = mi
                        v = plsc.load_gather(buf_v, [fb + iota * c_dim + ci])
                        take = v > m
                        return (jnp.where(take, v, m),
                                jnp.where(take, jnp.broadcast_to(ci, (LANES,)),
                                          idx))

                    m, idx = lax.fori_loop(0, c_dim, c_body, (neg_inf, zeros_i))
                    plsc.store_scatter(buf_v, [fb + iota * c_dim + idx], neg_inf)
                    plsc.store_scatter(idx_v, [rowcol + j], idx)
                return carry

        lax.fori_loop(0, groups, group_body, 0)
        pltpu.sync_copy(idx_v, idx_hbm.at[pl.ds(base * IDX_W, r_sub * IDX_W)])

    return sc_topk


def _make_tc_onehot_mm(n_out, npad, c_dim, rb):
    nb = npad // rb

    def body(idx_ref, tab_ref, out_ref):
        iota_c = lax.broadcasted_iota(jnp.int32, (rb, c_dim), 1)
        mask = jnp.zeros((rb, c_dim), jnp.float32)
        for j in range(K_SEL):
            mask += (iota_c == idx_ref[:, j:j + 1]).astype(jnp.float32)
        out_ref[...] = jnp.dot(mask, tab_ref[...],
                               preferred_element_type=jnp.float32)

    return pl.pallas_call(
        body,
        grid=(nb,),
        in_specs=[
            pl.BlockSpec((rb, IDX_W), lambda i: (i, 0)),
            pl.BlockSpec((c_dim, 512), lambda i: (0, 0)),
        ],
        out_specs=pl.BlockSpec((rb, 512), lambda i: (i, 0)),
        out_shape=jax.ShapeDtypeStruct((n_out, 512), jnp.float32),
    )


def _select(logits, table, npad, r_sub, rb):
    n, c_dim = logits.shape
    flat = jnp.pad(logits, ((0, npad - n), (0, 0))).reshape(-1)
    idx = _make_sc_topk(npad, c_dim, r_sub)(flat).reshape(npad, IDX_W)
    return _make_tc_onehot_mm(n, npad, c_dim, rb)(idx, table)


def kernel(node_logits, edge_logits, knode, kedge):
    knode_out = _select(node_logits, knode, npad=512, r_sub=16, rb=512)
    kedge_out = _select(edge_logits, kedge, npad=90112, r_sub=2816, rb=512)
    return (knode_out, kedge_out)


# speedup vs baseline: 8.0788x; 1.1567x over previous
import functools

import jax
import jax.numpy as jnp
from jax import lax
from jax.experimental import pallas as pl
from jax.experimental.pallas import tpu as pltpu
from jax.experimental.pallas import tpu_sc as plsc

K_SEL = 5
NC = 2
NS = 16
NW = NC * NS
LANES = 16
IDX_ROWS = 8


def _make_sc_topk(n, c_dim, r_sub, npad_idx):
    n_full = n // r_sub
    rem = n - n_full * r_sub
    rem_g = rem // LANES
    tail = rem - rem_g * LANES
    tail_groups = rem_g + (1 if tail else 0)
    rem_cnt = tail_groups * LANES

    mesh = plsc.VectorSubcoreMesh(core_axis_name="c", subcore_axis_name="s")

    @functools.partial(
        pl.kernel,
        out_type=jax.ShapeDtypeStruct((IDX_ROWS, npad_idx), jnp.int32),
        mesh=mesh,
        scratch_types=[
            pltpu.VMEM((r_sub, c_dim), jnp.float32),
            pltpu.VMEM((K_SEL, r_sub), jnp.int32),
        ],
        compiler_params=pltpu.CompilerParams(
            needs_layout_passes=False, use_tc_tiling_on_sc=False),
    )
    def sc_topk(logits_hbm, idx_hbm, buf_v, idx_v):
        wid = lax.axis_index("s") * NC + lax.axis_index("c")
        iota = lax.iota(jnp.int32, LANES)
        neg_inf = jnp.full((LANES,), -jnp.inf, dtype=jnp.float32)
        zeros_i = jnp.zeros((LANES,), jnp.int32)

        if c_dim <= 32:
            def group_body(g, carry):
                rows = g * LANES + iota
                vals = [plsc.load_gather(
                            buf_v, [rows, jnp.broadcast_to(jnp.int32(c),
                                                           (LANES,))])
                        for c in range(c_dim)]
                for j in range(K_SEL):
                    nodes = []
                    for c in range(0, c_dim - 1, 2):
                        take = vals[c + 1] > vals[c]
                        nodes.append((jnp.where(take, vals[c + 1], vals[c]),
                                      jnp.where(take, jnp.int32(c + 1),
                                                jnp.int32(c))))
                    if c_dim % 2:
                        nodes.append((vals[c_dim - 1],
                                      jnp.broadcast_to(jnp.int32(c_dim - 1),
                                                       (LANES,))))
                    while len(nodes) > 1:
                        nxt = []
                        for i in range(0, len(nodes) - 1, 2):
                            (va, ia), (vb, ib) = nodes[i], nodes[i + 1]
                            take = vb > va
                            nxt.append((jnp.where(take, vb, va),
                                        jnp.where(take, ib, ia)))
                        if len(nodes) % 2:
                            nxt.append(nodes[-1])
                        nodes = nxt
                    _, idx = nodes[0]
                    idx_v[j, pl.ds(g * LANES, LANES)] = idx
                    if j < K_SEL - 1:
                        vals = [jnp.where(idx == c, neg_inf, vals[c])
                                for c in range(c_dim)]
                return carry
        else:
            def group_body(g, carry):
                rows = g * LANES + iota
                for j in range(K_SEL):
                    def c_body(ci, mi):
                        m, idx = mi
                        v = plsc.load_gather(
                            buf_v, [rows, jnp.broadcast_to(ci, (LANES,))])
                        take = v > m
                        return (jnp.where(take, v, m),
                                jnp.where(take, jnp.broadcast_to(ci, (LANES,)),
                                          idx))

                    m, idx = lax.fori_loop(0, c_dim, c_body,
                                           (neg_inf, zeros_i))
                    plsc.store_scatter(buf_v, [rows, idx], neg_inf)
                    idx_v[j, pl.ds(g * LANES, LANES)] = idx
                return carry

        @pl.when(wid < n_full)
        def _full_chunk():
            base = wid * r_sub
            pltpu.sync_copy(logits_hbm.at[pl.ds(base, r_sub)], buf_v)
            lax.fori_loop(0, r_sub // LANES, group_body, 0)
            for j in range(K_SEL):
                pltpu.sync_copy(idx_v.at[j],
                                idx_hbm.at[j, pl.ds(base, r_sub)])

        if rem > 0:
            @pl.when(wid == n_full)
            def _tail_chunk():
                base = n_full * r_sub
                if rem_g:
                    pltpu.sync_copy(
                        logits_hbm.at[pl.ds(base, rem_g * LANES)],
                        buf_v.at[pl.ds(0, rem_g * LANES)])
                if tail:
                    pltpu.sync_copy(
                        logits_hbm.at[pl.ds(base + rem_g * LANES, tail)],
                        buf_v.at[pl.ds(rem_g * LANES, tail)])
                lax.fori_loop(0, tail_groups, group_body, 0)
                for j in range(K_SEL):
                    pltpu.sync_copy(idx_v.at[j, pl.ds(0, rem_cnt)],
                                    idx_hbm.at[j, pl.ds(base, rem_cnt)])

    return sc_topk


def _make_tc_onehot_mm(n_out, c_dim, rb, nb):
    def body(idx_ref, tab_ref, out_ref):
        iota_ct = lax.broadcasted_iota(jnp.int32, (c_dim, rb), 0)
        hit = iota_ct == idx_ref[0:1, :]
        for j in range(1, K_SEL):
            hit = jnp.logical_or(hit, iota_ct == idx_ref[j:j + 1, :])
        mask_t = jnp.where(hit, jnp.float32(1.0), jnp.float32(0.0))
        out_ref[...] = lax.dot_general(
            mask_t, tab_ref[...],
            dimension_numbers=(((0,), (0,)), ((), ())),
            preferred_element_type=jnp.float32)

    return pl.pallas_call(
        body,
        grid=(nb,),
        in_specs=[
            pl.BlockSpec((IDX_ROWS, rb), lambda i: (0, i)),
            pl.BlockSpec((c_dim, 512), lambda i: (0, 0)),
        ],
        out_specs=pl.BlockSpec((rb, 512), lambda i: (i, 0)),
        out_shape=jax.ShapeDtypeStruct((n_out, 512), jnp.float32),
    )


def _select(logits, table, r_sub, rb, nb, npad_idx):
    n, c_dim = logits.shape
    idx_t = _make_sc_topk(n, c_dim, r_sub, npad_idx)(logits)
    return _make_tc_onehot_mm(n, c_dim, rb, nb)(idx_t, table)


def kernel(node_logits, edge_logits, knode, kedge):
    knode_out = _select(node_logits, knode, r_sub=16, rb=304, nb=1,
                        npad_idx=304)
    kedge_out = _select(edge_logits, kedge, r_sub=2816, rb=512, nb=176,
                        npad_idx=90112)
    return (knode_out, kedge_out)


# speedup vs baseline: 9.2681x; 1.1472x over previous
import functools

import jax
import jax.numpy as jnp
from jax import lax
from jax.experimental import pallas as pl
from jax.experimental.pallas import tpu as pltpu
from jax.experimental.pallas import tpu_sc as plsc

K_SEL = 5
NC = 2
NS = 16
NW = NC * NS
LANES = 16
IDX_ROWS = 8


def _tree_top5(vals, c_dim, neg_inf, emit):
    for j in range(K_SEL):
        nodes = []
        for c in range(0, c_dim - 1, 2):
            take = vals[c + 1] > vals[c]
            nodes.append((jnp.where(take, vals[c + 1], vals[c]),
                          jnp.where(take, jnp.int32(c + 1), jnp.int32(c))))
        if c_dim % 2:
            nodes.append((vals[c_dim - 1],
                          jnp.broadcast_to(jnp.int32(c_dim - 1), (LANES,))))
        while len(nodes) > 1:
            nxt = []
            for i in range(0, len(nodes) - 1, 2):
                (va, ia), (vb, ib) = nodes[i], nodes[i + 1]
                take = vb > va
                nxt.append((jnp.where(take, vb, va),
                            jnp.where(take, ib, ia)))
            if len(nodes) % 2:
                nxt.append(nodes[-1])
            nodes = nxt
        _, idx = nodes[0]
        emit(j, idx)
        if j < K_SEL - 1:
            vals = [jnp.where(idx == c, neg_inf, vals[c])
                    for c in range(c_dim)]


def _make_sc_topk_edge(n, c_dim, r_sub, r_chunk, npad_idx):
    n_full = n // r_sub
    rem = n - n_full * r_sub
    rem_g = rem // LANES
    tail = rem - rem_g * LANES
    rem_cnt = -(-(rem_g * LANES + (LANES if tail else 0)) // 128) * 128
    n_chunks = r_sub // r_chunk
    rem_full_chunks = rem_g * LANES // r_chunk
    rem_last_g = rem_g - rem_full_chunks * (r_chunk // LANES)

    mesh = plsc.VectorSubcoreMesh(core_axis_name="c", subcore_axis_name="s")

    @functools.partial(
        pl.kernel,
        out_type=jax.ShapeDtypeStruct((IDX_ROWS, npad_idx), jnp.int32),
        mesh=mesh,
        scratch_types=[
            pltpu.VMEM((r_chunk, c_dim), jnp.float32),
            pltpu.VMEM((IDX_ROWS, r_sub), jnp.int32),
        ],
        compiler_params=pltpu.CompilerParams(
            needs_layout_passes=False, use_tc_tiling_on_sc=True),
    )
    def sc_topk(logits_hbm, tail16_hbm, idx_hbm, buf_v, idx_v):
        wid = lax.axis_index("s") * NC + lax.axis_index("c")
        iota = lax.iota(jnp.int32, LANES)
        neg_inf = jnp.full((LANES,), -jnp.inf, dtype=jnp.float32)

        def group_at(buf_row, idx_col):
            rows = buf_row + iota
            vals = [plsc.load_gather(
                        buf_v, [rows,
                                jnp.broadcast_to(jnp.int32(c), (LANES,))])
                    for c in range(c_dim)]

            def emit(j, idx):
                idx_v[j, pl.ds(idx_col, LANES)] = idx

            _tree_top5(vals, c_dim, neg_inf, emit)

        def chunk_groups(ch, n_groups):
            def group_body(g, carry):
                group_at(g * LANES, ch * r_chunk + g * LANES)
                return carry
            lax.fori_loop(0, n_groups, group_body, 0)

        @pl.when(wid < n_full)
        def _full_chunk():
            base = wid * r_sub

            def chunk_body(ch, carry):
                pltpu.sync_copy(
                    logits_hbm.at[pl.ds(base + ch * r_chunk, r_chunk)], buf_v)
                chunk_groups(ch, r_chunk // LANES)
                return carry

            lax.fori_loop(0, n_chunks, chunk_body, 0)
            pltpu.sync_copy(idx_v, idx_hbm.at[:, pl.ds(base, r_sub)])

        if rem > 0:
            @pl.when(wid == n_full)
            def _tail_chunk():
                base = n_full * r_sub

                def chunk_body(ch, carry):
                    pltpu.sync_copy(
                        logits_hbm.at[pl.ds(base + ch * r_chunk, r_chunk)],
                        buf_v)
                    chunk_groups(ch, r_chunk // LANES)
                    return carry

                lax.fori_loop(0, rem_full_chunks, chunk_body, 0)
                done = rem_full_chunks * r_chunk
                if rem_last_g:
                    pltpu.sync_copy(
                        logits_hbm.at[pl.ds(base + done, rem_last_g * LANES)],
                        buf_v.at[pl.ds(0, rem_last_g * LANES)])
                    chunk_groups(rem_full_chunks, rem_last_g)
                if tail:
                    pltpu.sync_copy(
                        tail16_hbm,
                        buf_v.at[pl.ds(rem_last_g * LANES, LANES)])
                    group_at(rem_last_g * LANES, rem - LANES)
                pltpu.sync_copy(
                    idx_v.at[:, pl.ds(0, rem_cnt)],
                    idx_hbm.at[:, pl.ds(base, rem_cnt)])

    return sc_topk


def _make_sc_topk_node(n, c_dim, r_sub, npad_idx):
    n_full = n // r_sub
    rem = n - n_full * r_sub
    rem_g = rem // LANES
    tail = rem - rem_g * LANES
    tail_groups = rem_g + (1 if tail else 0)
    rem_cnt = tail_groups * LANES

    mesh = plsc.VectorSubcoreMesh(core_axis_name="c", subcore_axis_name="s")

    @functools.partial(
        pl.kernel,
        out_type=jax.ShapeDtypeStruct((IDX_ROWS, npad_idx), jnp.int32),
        mesh=mesh,
        scratch_types=[
            pltpu.VMEM((r_sub, c_dim), jnp.float32),
            pltpu.VMEM((K_SEL, r_sub), jnp.int32),
        ],
        compiler_params=pltpu.CompilerParams(
            needs_layout_passes=False, use_tc_tiling_on_sc=False),
    )
    def sc_topk(logits_hbm, idx_hbm, buf_v, idx_v):
        wid = lax.axis_index("s") * NC + lax.axis_index("c")
        iota = lax.iota(jnp.int32, LANES)
        neg_inf = jnp.full((LANES,), -jnp.inf, dtype=jnp.float32)
        zeros_i = jnp.zeros((LANES,), jnp.int32)

        def group_body(g, carry):
            rows = g * LANES + iota
            for j in range(K_SEL):
                def c_body(ci, mi):
                    m, idx = mi
                    v = plsc.load_gather(
                        buf_v, [rows, jnp.broadcast_to(ci, (LANES,))])
                    take = v > m
                    return (jnp.where(take, v, m),
                            jnp.where(take, jnp.broadcast_to(ci, (LANES,)),
                                      idx))

                m, idx = lax.fori_loop(0, c_dim, c_body, (neg_inf, zeros_i))
                plsc.store_scatter(buf_v, [rows, idx], neg_inf)
                idx_v[j, pl.ds(g * LANES, LANES)] = idx
            return carry

        @pl.when(wid < n_full)
        def _full_chunk():
            base = wid * r_sub
            pltpu.sync_copy(logits_hbm.at[pl.ds(base, r_sub)], buf_v)
            lax.fori_loop(0, r_sub // LANES, group_body, 0)
            for j in range(K_SEL):
                pltpu.sync_copy(idx_v.at[pl.ds(j, 1)],
                                idx_hbm.at[pl.ds(j, 1), pl.ds(base, r_sub)])

        if rem > 0:
            @pl.when(wid == n_full)
            def _tail_chunk():
                base = n_full * r_sub
                if rem_g:
                    pltpu.sync_copy(
                        logits_hbm.at[pl.ds(base, rem_g * LANES)],
                        buf_v.at[pl.ds(0, rem_g * LANES)])
                if tail:
                    pltpu.sync_copy(
                        logits_hbm.at[pl.ds(base + rem_g * LANES, tail)],
                        buf_v.at[pl.ds(rem_g * LANES, tail)])
                lax.fori_loop(0, tail_groups, group_body, 0)
                for j in range(K_SEL):
                    pltpu.sync_copy(
                        idx_v.at[pl.ds(j, 1), pl.ds(0, rem_cnt)],
                        idx_hbm.at[pl.ds(j, 1), pl.ds(base, rem_cnt)])

    return sc_topk


def _make_tc_onehot_mm(n_out, c_dim, rb, nb):
    def body(idx_ref, tab_ref, out_ref):
        iota_ct = lax.broadcasted_iota(jnp.int32, (c_dim, rb), 0)
        hit = iota_ct == idx_ref[0:1, :]
        for j in range(1, K_SEL):
            hit = jnp.logical_or(hit, iota_ct == idx_ref[j:j + 1, :])
        mask_t = jnp.where(hit, jnp.float32(1.0), jnp.float32(0.0))
        out_ref[...] = lax.dot_general(
            mask_t, tab_ref[...],
            dimension_numbers=(((0,), (0,)), ((), ())),
            preferred_element_type=jnp.float32)

    return pl.pallas_call(
        body,
        grid=(nb,),
        in_specs=[
            pl.BlockSpec((IDX_ROWS, rb), lambda i: (0, i)),
            pl.BlockSpec((c_dim, 512), lambda i: (0, 0)),
        ],
        out_specs=pl.BlockSpec((rb, 512), lambda i: (i, 0)),
        out_shape=jax.ShapeDtypeStruct((n_out, 512), jnp.float32),
    )


def kernel(node_logits, edge_logits, knode, kedge):
    nn, nc = node_logits.shape
    node_idx = _make_sc_topk_node(nn, nc, r_sub=16, npad_idx=304)(node_logits)
    knode_out = _make_tc_onehot_mm(nn, nc, rb=304, nb=1)(node_idx, knode)

    ne, ce = edge_logits.shape
    tail16 = lax.slice(edge_logits, (ne - LANES, 0), (ne, ce))
    edge_idx = _make_sc_topk_edge(ne, ce, r_sub=2816, r_chunk=704,
                                  npad_idx=90112)(edge_logits, tail16)
    kedge_out = _make_tc_onehot_mm(ne, ce, rb=512, nb=176)(edge_idx, kedge)
    return (knode_out, kedge_out)


# speedup vs baseline: 11.7237x; 1.2650x over previous
import functools

import jax
import jax.numpy as jnp
from jax import lax
from jax.experimental import pallas as pl
from jax.experimental.pallas import tpu as pltpu
from jax.experimental.pallas import tpu_sc as plsc

K_SEL = 5
NC = 2
NS = 16
NW = NC * NS
LANES = 16
IDX_ROWS = 8


def _tree_top5(vals, c_dim, neg_inf, emit):
    for j in range(K_SEL):
        nodes = []
        for c in range(0, c_dim - 1, 2):
            take = vals[c + 1] > vals[c]
            nodes.append((jnp.where(take, vals[c + 1], vals[c]),
                          jnp.where(take, jnp.int32(c + 1), jnp.int32(c))))
        if c_dim % 2:
            nodes.append((vals[c_dim - 1],
                          jnp.broadcast_to(jnp.int32(c_dim - 1), (LANES,))))
        while len(nodes) > 1:
            nxt = []
            for i in range(0, len(nodes) - 1, 2):
                (va, ia), (vb, ib) = nodes[i], nodes[i + 1]
                take = vb > va
                nxt.append((jnp.where(take, vb, va),
                            jnp.where(take, ib, ia)))
            if len(nodes) % 2:
                nxt.append(nodes[-1])
            nodes = nxt
        _, idx = nodes[0]
        emit(j, idx)
        if j < K_SEL - 1:
            vals = [jnp.where(idx == c, neg_inf, vals[c])
                    for c in range(c_dim)]


def _make_sc_topk_edge(n, c_dim, r_sub, npad_idx):
    n_full = n // r_sub
    rem = n - n_full * r_sub
    rem_al = (rem // 128) * 128
    tail_g = (rem - rem_al + 127) // 128 * 8
    rem_cnt = -(-rem // 128) * 128

    mesh = plsc.VectorSubcoreMesh(core_axis_name="c", subcore_axis_name="s")

    @functools.partial(
        pl.kernel,
        out_type=jax.ShapeDtypeStruct((IDX_ROWS, npad_idx), jnp.int32),
        mesh=mesh,
        scratch_types=[
            pltpu.VMEM((c_dim, r_sub), jnp.float32),
            pltpu.VMEM((IDX_ROWS, r_sub), jnp.int32),
        ],
        compiler_params=pltpu.CompilerParams(
            needs_layout_passes=False, use_tc_tiling_on_sc=True),
    )
    def sc_topk(logits_t_hbm, tail_t_hbm, idx_hbm, buf_v, idx_v):
        wid = lax.axis_index("s") * NC + lax.axis_index("c")
        iota = lax.iota(jnp.int32, LANES)
        neg_inf = jnp.full((LANES,), -jnp.inf, dtype=jnp.float32)

        def group_at(buf_col, idx_col):
            cols = buf_col + iota
            vals = [plsc.load_gather(
                        buf_v, [jnp.broadcast_to(jnp.int32(c), (LANES,)),
                                cols])
                    for c in range(c_dim)]

            def emit(j, idx):
                idx_v[j, pl.ds(idx_col, LANES)] = idx

            _tree_top5(vals, c_dim, neg_inf, emit)

        @pl.when(wid < n_full)
        def _full_chunk():
            base = wid * r_sub
            pltpu.sync_copy(logits_t_hbm.at[:, pl.ds(base, r_sub)], buf_v)

            def group_body(g, carry):
                group_at(g * LANES, g * LANES)
                return carry

            lax.fori_loop(0, r_sub // LANES, group_body, 0)
            pltpu.sync_copy(idx_v, idx_hbm.at[:, pl.ds(base, r_sub)])

        if rem > 0:
            @pl.when(wid == n_full)
            def _tail_chunk():
                base = n_full * r_sub
                if rem_al:
                    pltpu.sync_copy(
                        logits_t_hbm.at[:, pl.ds(base, rem_al)],
                        buf_v.at[:, pl.ds(0, rem_al)])

                    def group_body(g, carry):
                        group_at(g * LANES, g * LANES)
                        return carry

                    lax.fori_loop(0, rem_al // LANES, group_body, 0)
                pltpu.sync_copy(tail_t_hbm,
                                buf_v.at[:, pl.ds(rem_al, 128)])

                def tail_body(g, carry):
                    group_at(rem_al + g * LANES, rem - 128 + g * LANES)
                    return carry

                lax.fori_loop(0, tail_g, tail_body, 0)
                pltpu.sync_copy(
                    idx_v.at[:, pl.ds(0, rem_cnt)],
                    idx_hbm.at[:, pl.ds(base, rem_cnt)])

    return sc_topk


def _make_sc_topk_node(n, c_dim, r_sub, npad_idx):
    n_full = n // r_sub
    rem = n - n_full * r_sub
    rem_g = rem // LANES
    tail = rem - rem_g * LANES
    tail_groups = rem_g + (1 if tail else 0)
    rem_cnt = tail_groups * LANES

    mesh = plsc.VectorSubcoreMesh(core_axis_name="c", subcore_axis_name="s")

    @functools.partial(
        pl.kernel,
        out_type=jax.ShapeDtypeStruct((IDX_ROWS, npad_idx), jnp.int32),
        mesh=mesh,
        scratch_types=[
            pltpu.VMEM((r_sub, c_dim), jnp.float32),
            pltpu.VMEM((K_SEL, r_sub), jnp.int32),
        ],
        compiler_params=pltpu.CompilerParams(
            needs_layout_passes=False, use_tc_tiling_on_sc=False),
    )
    def sc_topk(logits_hbm, idx_hbm, buf_v, idx_v):
        wid = lax.axis_index("s") * NC + lax.axis_index("c")
        iota = lax.iota(jnp.int32, LANES)
        neg_inf = jnp.full((LANES,), -jnp.inf, dtype=jnp.float32)
        zeros_i = jnp.zeros((LANES,), jnp.int32)

        def group_body(g, carry):
            rows = g * LANES + iota
            for j in range(K_SEL):
                def c_body(ci, mi):
                    m, idx = mi
                    v = plsc.load_gather(
                        buf_v, [rows, jnp.broadcast_to(ci, (LANES,))])
                    take = v > m
                    return (jnp.where(take, v, m),
                            jnp.where(take, jnp.broadcast_to(ci, (LANES,)),
                                      idx))

                m, idx = lax.fori_loop(0, c_dim, c_body, (neg_inf, zeros_i))
                plsc.store_scatter(buf_v, [rows, idx], neg_inf)
                idx_v[j, pl.ds(g * LANES, LANES)] = idx
            return carry

        @pl.when(wid < n_full)
        def _full_chunk():
            base = wid * r_sub
            pltpu.sync_copy(logits_hbm.at[pl.ds(base, r_sub)], buf_v)
            lax.fori_loop(0, r_sub // LANES, group_body, 0)
            for j in range(K_SEL):
                pltpu.sync_copy(idx_v.at[pl.ds(j, 1)],
                                idx_hbm.at[pl.ds(j, 1), pl.ds(base, r_sub)])

        if rem > 0:
            @pl.when(wid == n_full)
            def _tail_chunk():
                base = n_full * r_sub
                if rem_g:
                    pltpu.sync_copy(
                        logits_hbm.at[pl.ds(base, rem_g * LANES)],
                        buf_v.at[pl.ds(0, rem_g * LANES)])
                if tail:
                    pltpu.sync_copy(
                        logits_hbm.at[pl.ds(base + rem_g * LANES, tail)],
                        buf_v.at[pl.ds(rem_g * LANES, tail)])
                lax.fori_loop(0, tail_groups, group_body, 0)
                for j in range(K_SEL):
                    pltpu.sync_copy(
                        idx_v.at[pl.ds(j, 1), pl.ds(0, rem_cnt)],
                        idx_hbm.at[pl.ds(j, 1), pl.ds(base, rem_cnt)])

    return sc_topk


def _make_tc_onehot_mm(n_out, c_dim, rb, nb):
    def body(idx_ref, tab_ref, out_ref):
        iota_ct = lax.broadcasted_iota(jnp.int32, (c_dim, rb), 0)
        hit = iota_ct == idx_ref[0:1, :]
        for j in range(1, K_SEL):
            hit = jnp.logical_or(hit, iota_ct == idx_ref[j:j + 1, :])
        mask_t = jnp.where(hit, jnp.float32(1.0), jnp.float32(0.0))
        out_ref[...] = lax.dot_general(
            mask_t, tab_ref[...],
            dimension_numbers=(((0,), (0,)), ((), ())),
            preferred_element_type=jnp.float32)

    return pl.pallas_call(
        body,
        grid=(nb,),
        in_specs=[
            pl.BlockSpec((IDX_ROWS, rb), lambda i: (0, i)),
            pl.BlockSpec((c_dim, 512), lambda i: (0, 0)),
        ],
        out_specs=pl.BlockSpec((rb, 512), lambda i: (i, 0)),
        out_shape=jax.ShapeDtypeStruct((n_out, 512), jnp.float32),
    )


def kernel(node_logits, edge_logits, knode, kedge):
    nn, nc = node_logits.shape
    node_idx = _make_sc_topk_node(nn, nc, r_sub=16, npad_idx=304)(node_logits)
    knode_out = _make_tc_onehot_mm(nn, nc, rb=304, nb=1)(node_idx, knode)

    ne, ce = edge_logits.shape
    logits_t = edge_logits.T
    tail_t = lax.slice(logits_t, (0, ne - 128), (ce, ne))
    edge_idx = _make_sc_topk_edge(ne, ce, r_sub=2816, npad_idx=90112)(
        logits_t, tail_t)
    kedge_out = _make_tc_onehot_mm(ne, ce, rb=512, nb=176)(edge_idx, kedge)
    return (knode_out, kedge_out)


# speedup vs baseline: 11.7365x; 1.0011x over previous
import functools

import jax
import jax.numpy as jnp
from jax import lax
from jax.experimental import pallas as pl
from jax.experimental.pallas import tpu as pltpu
from jax.experimental.pallas import tpu_sc as plsc

K_SEL = 5
NC = 2
NS = 16
NW = NC * NS
LANES = 16
IDX_ROWS = 8


def _tree_top5(vals, c_dim, neg_inf, emit):
    for j in range(K_SEL):
        nodes = []
        for c in range(0, c_dim - 1, 2):
            take = vals[c + 1] > vals[c]
            nodes.append((jnp.where(take, vals[c + 1], vals[c]),
                          jnp.where(take, jnp.int32(c + 1), jnp.int32(c))))
        if c_dim % 2:
            nodes.append((vals[c_dim - 1],
                          jnp.broadcast_to(jnp.int32(c_dim - 1), (LANES,))))
        while len(nodes) > 1:
            nxt = []
            for i in range(0, len(nodes) - 1, 2):
                (va, ia), (vb, ib) = nodes[i], nodes[i + 1]
                take = vb > va
                nxt.append((jnp.where(take, vb, va),
                            jnp.where(take, ib, ia)))
            if len(nodes) % 2:
                nxt.append(nodes[-1])
            nodes = nxt
        _, idx = nodes[0]
        emit(j, idx)
        if j < K_SEL - 1:
            vals = [jnp.where(idx == c, neg_inf, vals[c])
                    for c in range(c_dim)]


def _make_sc_topk_edge(n, c_dim, r_sub, npad_idx):
    n_full = n // r_sub
    rem = n - n_full * r_sub
    rem_al = (rem // 128) * 128
    tail_g = (-(-rem // LANES) * LANES - rem_al) // LANES
    tail_off = 128 - (rem - rem_al)
    rem_cnt = -(-rem // 128) * 128

    mesh = plsc.VectorSubcoreMesh(core_axis_name="c", subcore_axis_name="s")

    @functools.partial(
        pl.kernel,
        out_type=jax.ShapeDtypeStruct((IDX_ROWS, npad_idx), jnp.int32),
        mesh=mesh,
        scratch_types=[
            pltpu.VMEM((c_dim, r_sub), jnp.float32),
            pltpu.VMEM((IDX_ROWS, r_sub), jnp.int32),
        ],
        compiler_params=pltpu.CompilerParams(
            needs_layout_passes=False, use_tc_tiling_on_sc=True),
    )
    def sc_topk(logits_t_hbm, tail_t_hbm, idx_hbm, buf_v, idx_v):
        wid = lax.axis_index("s") * NC + lax.axis_index("c")
        iota = lax.iota(jnp.int32, LANES)
        neg_inf = jnp.full((LANES,), -jnp.inf, dtype=jnp.float32)

        def group_at(buf_col, idx_col):
            cols = buf_col + iota
            vals = [plsc.load_gather(
                        buf_v, [jnp.broadcast_to(jnp.int32(c), (LANES,)),
                                cols])
                    for c in range(c_dim)]

            def emit(j, idx):
                idx_v[j, pl.ds(idx_col, LANES)] = idx

            _tree_top5(vals, c_dim, neg_inf, emit)

        @pl.when(wid < n_full)
        def _full_chunk():
            base = wid * r_sub
            pltpu.sync_copy(logits_t_hbm.at[:, pl.ds(base, r_sub)], buf_v)

            def group_body(g, carry):
                group_at(g * LANES, g * LANES)
                return carry

            lax.fori_loop(0, r_sub // LANES, group_body, 0)
            pltpu.sync_copy(idx_v, idx_hbm.at[:, pl.ds(base, r_sub)])

        if rem > 0:
            @pl.when(wid == n_full)
            def _tail_chunk():
                base = n_full * r_sub
                if rem_al:
                    pltpu.sync_copy(
                        logits_t_hbm.at[:, pl.ds(base, rem_al)],
                        buf_v.at[:, pl.ds(0, rem_al)])

                    def group_body(g, carry):
                        group_at(g * LANES, g * LANES)
                        return carry

                    lax.fori_loop(0, rem_al // LANES, group_body, 0)
                pltpu.sync_copy(tail_t_hbm,
                                buf_v.at[:, pl.ds(rem_al, 128)])

                def tail_body(g, carry):
                    group_at(rem_al + tail_off + g * LANES,
                             rem_al + g * LANES)
                    return carry

                lax.fori_loop(0, tail_g, tail_body, 0)
                pltpu.sync_copy(
                    idx_v.at[:, pl.ds(0, rem_cnt)],
                    idx_hbm.at[:, pl.ds(base, rem_cnt)])

    return sc_topk


def _make_sc_topk_node(n, c_dim, r_sub, npad_idx):
    n_full = n // r_sub
    rem = n - n_full * r_sub
    rem_g = rem // LANES
    tail = rem - rem_g * LANES
    tail_groups = rem_g + (1 if tail else 0)
    rem_cnt = tail_groups * LANES

    mesh = plsc.VectorSubcoreMesh(core_axis_name="c", subcore_axis_name="s")

    @functools.partial(
        pl.kernel,
        out_type=jax.ShapeDtypeStruct((IDX_ROWS, npad_idx), jnp.int32),
        mesh=mesh,
        scratch_types=[
            pltpu.VMEM((r_sub, c_dim), jnp.float32),
            pltpu.VMEM((K_SEL, r_sub), jnp.int32),
        ],
        compiler_params=pltpu.CompilerParams(
            needs_layout_passes=False, use_tc_tiling_on_sc=False),
    )
    def sc_topk(logits_hbm, idx_hbm, buf_v, idx_v):
        wid = lax.axis_index("s") * NC + lax.axis_index("c")
        iota = lax.iota(jnp.int32, LANES)
        neg_inf = jnp.full((LANES,), -jnp.inf, dtype=jnp.float32)
        zeros_i = jnp.zeros((LANES,), jnp.int32)

        def group_body(g, carry):
            rows = g * LANES + iota
            for j in range(K_SEL):
                def c_body(ci, mi):
                    m, idx = mi
                    v = plsc.load_gather(
                        buf_v, [rows, jnp.broadcast_to(ci, (LANES,))])
                    take = v > m
                    return (jnp.where(take, v, m),
                            jnp.where(take, jnp.broadcast_to(ci, (LANES,)),
                                      idx))

                m, idx = lax.fori_loop(0, c_dim, c_body, (neg_inf, zeros_i))
                plsc.store_scatter(buf_v, [rows, idx], neg_inf)
                idx_v[j, pl.ds(g * LANES, LANES)] = idx
            return carry

        @pl.when(wid < n_full)
        def _full_chunk():
            base = wid * r_sub
            pltpu.sync_copy(logits_hbm.at[pl.ds(base, r_sub)], buf_v)
            lax.fori_loop(0, r_sub // LANES, group_body, 0)
            for j in range(K_SEL):
                pltpu.sync_copy(idx_v.at[pl.ds(j, 1)],
                                idx_hbm.at[pl.ds(j, 1), pl.ds(base, r_sub)])

        if rem > 0:
            @pl.when(wid == n_full)
            def _tail_chunk():
                base = n_full * r_sub
                if rem_g:
                    pltpu.sync_copy(
                        logits_hbm.at[pl.ds(base, rem_g * LANES)],
                        buf_v.at[pl.ds(0, rem_g * LANES)])
                if tail:
                    pltpu.sync_copy(
                        logits_hbm.at[pl.ds(base + rem_g * LANES, tail)],
                        buf_v.at[pl.ds(rem_g * LANES, tail)])
                lax.fori_loop(0, tail_groups, group_body, 0)
                for j in range(K_SEL):
                    pltpu.sync_copy(
                        idx_v.at[pl.ds(j, 1), pl.ds(0, rem_cnt)],
                        idx_hbm.at[pl.ds(j, 1), pl.ds(base, rem_cnt)])

    return sc_topk


def _make_tc_onehot_mm(n_out, c_dim, rb, nb):
    def body(idx_ref, tab_ref, out_ref):
        iota_ct = lax.broadcasted_iota(jnp.int32, (c_dim, rb), 0)
        hit = iota_ct == idx_ref[0:1, :]
        for j in range(1, K_SEL):
            hit = jnp.logical_or(hit, iota_ct == idx_ref[j:j + 1, :])
        mask_t = jnp.where(hit, jnp.float32(1.0), jnp.float32(0.0))
        out_ref[...] = lax.dot_general(
            mask_t, tab_ref[...],
            dimension_numbers=(((0,), (0,)), ((), ())),
            preferred_element_type=jnp.float32)

    return pl.pallas_call(
        body,
        grid=(nb,),
        in_specs=[
            pl.BlockSpec((IDX_ROWS, rb), lambda i: (0, i)),
            pl.BlockSpec((c_dim, 512), lambda i: (0, 0)),
        ],
        out_specs=pl.BlockSpec((rb, 512), lambda i: (i, 0)),
        out_shape=jax.ShapeDtypeStruct((n_out, 512), jnp.float32),
    )


def kernel(node_logits, edge_logits, knode, kedge):
    nn, nc = node_logits.shape
    node_idx = _make_sc_topk_node(nn, nc, r_sub=16, npad_idx=304)(node_logits)
    knode_out = _make_tc_onehot_mm(nn, nc, rb=304, nb=1)(node_idx, knode)

    ne, ce = edge_logits.shape
    logits_t = edge_logits.T
    tail_t = lax.slice(logits_t, (0, ne - 128), (ce, ne))
    edge_idx = _make_sc_topk_edge(ne, ce, r_sub=2816, npad_idx=90112)(
        logits_t, tail_t)
    kedge_out = _make_tc_onehot_mm(ne, ce, rb=512, nb=176)(edge_idx, kedge)
    return (knode_out, kedge_out)
